# Initial kernel scaffold; baseline (speedup 1.0000x reference)
import functools

import jax
import jax.numpy as jnp
from jax import lax
from jax.experimental import pallas as pl
from jax.experimental.pallas import tpu as pltpu
from jax.experimental.pallas import tpu_sc as plsc

_N = 10000
_E = 320000
_D_IN = 128
_D_H = 128
_D_OUT = 64

_NC = 2
_NS = 16
_NW = _NC * _NS

_N_PAD = 10240
_EROWS = 2560
_E_PAD = _EROWS * 128
_EROWS_W = _EROWS // _NW
_NROWS_S = _N_PAD // _NS

_BM = 1024
_GRID = _N_PAD // _BM

_mesh = plsc.VectorSubcoreMesh(core_axis_name="c", subcore_axis_name="s")


def _make_deg():
  @functools.partial(
      pl.kernel,
      out_type=(
          jax.ShapeDtypeStruct((_NC, _N_PAD, 8), jnp.float32),
          jax.ShapeDtypeStruct((_NC, _N_PAD, 8), jnp.float32),
      ),
      mesh=_mesh,
      scratch_types=[
          pltpu.VMEM((_EROWS_W, 128), jnp.int32),
          pltpu.VMEM((_EROWS_W, 128), jnp.int32),
          pltpu.VMEM((128, 8), jnp.float32),
          pltpu.VMEM_SHARED((_N_PAD, 8), jnp.float32),
          pltpu.VMEM_SHARED((_N_PAD, 8), jnp.float32),
      ],
  )
  def deg(src_hbm, dst_hbm, ones_hbm, zeros_hbm, do_hbm, di_hbm,
          sidx, didx, ones, acco, acci):
    c = lax.axis_index("c")
    s = lax.axis_index("s")
    w = s * _NC + c
    r0 = s * _NROWS_S
    pltpu.sync_copy(zeros_hbm.at[pl.ds(r0, _NROWS_S)], acco.at[pl.ds(r0, _NROWS_S)])
    pltpu.sync_copy(zeros_hbm.at[pl.ds(r0, _NROWS_S)], acci.at[pl.ds(r0, _NROWS_S)])
    pltpu.sync_copy(ones_hbm, ones)
    row0 = w * _EROWS_W
    pltpu.sync_copy(src_hbm.at[pl.ds(row0, _EROWS_W)], sidx)
    pltpu.sync_copy(dst_hbm.at[pl.ds(row0, _EROWS_W)], didx)
    plsc.subcore_barrier()

    @pl.loop(0, _EROWS_W // 8)
    def _(i):
      for j in range(8):
        pltpu.sync_copy(ones, acco.at[sidx.at[i * 8 + j]], add=True)
        pltpu.sync_copy(ones, acci.at[didx.at[i * 8 + j]], add=True)

    plsc.subcore_barrier()
    pltpu.sync_copy(acco.at[pl.ds(r0, _NROWS_S)], do_hbm.at[c, pl.ds(r0, _NROWS_S)])
    pltpu.sync_copy(acci.at[pl.ds(r0, _NROWS_S)], di_hbm.at[c, pl.ds(r0, _NROWS_S)])

  return deg


def _make_prop(d):
  @functools.partial(
      pl.kernel,
      out_type=jax.ShapeDtypeStruct((_NC, _N_PAD, d), jnp.float32),
      mesh=_mesh,
      scratch_types=[
          pltpu.VMEM((8, 128), jnp.int32),
          pltpu.VMEM((8, 128), jnp.int32),
          pltpu.VMEM((128, d), jnp.float32),
          pltpu.VMEM_SHARED((_N_PAD, d), jnp.float32),
          pltpu.SemaphoreType.DMA,
      ],
  )
  def prop(src_hbm, dst_hbm, xw_hbm, zeros_hbm, out_hbm,
           sidx, didx, rows, acc, sem):
    c = lax.axis_index("c")
    s = lax.axis_index("s")
    w = s * _NC + c
    r0 = s * _NROWS_S
    pltpu.sync_copy(zeros_hbm.at[pl.ds(r0, _NROWS_S)], acc.at[pl.ds(r0, _NROWS_S)])
    plsc.subcore_barrier()
    row0 = w * _EROWS_W

    @pl.loop(0, _EROWS_W // 8)
    def _(i):
      base = row0 + i * 8
      pltpu.sync_copy(src_hbm.at[pl.ds(base, 8)], sidx)
      pltpu.sync_copy(dst_hbm.at[pl.ds(base, 8)], didx)
      for j in range(8):
        pltpu.async_copy(xw_hbm.at[sidx.at[j]], rows, sem).wait()
        pltpu.sync_copy(rows, acc.at[didx.at[j]], add=True)

    plsc.subcore_barrier()
    pltpu.sync_copy(acc.at[pl.ds(r0, _NROWS_S)], out_hbm.at[c, pl.ds(r0, _NROWS_S)])

  return prop


_deg = _make_deg()
_prop_h = _make_prop(_D_H)
_prop_o = _make_prop(_D_OUT)


def _scale_from(deg_ref):
  dsum = deg_ref[0][:, 0:1] + deg_ref[1][:, 0:1]
  return lax.rsqrt(jnp.maximum(dsum, 1.0))


def _mm1_body(x_ref, w_ref, dego_ref, o_ref):
  xw = jnp.dot(x_ref[...], w_ref[...], preferred_element_type=jnp.float32)
  o_ref[...] = _scale_from(dego_ref) * xw


def _mid_body(p_ref, degi_ref, dego_ref, b1_ref, w2_ref, h1_ref, xw2_ref):
  agg = (p_ref[0] + p_ref[1]) * _scale_from(degi_ref)
  h1 = jnp.maximum(agg + b1_ref[...], 0.0)
  h1_ref[...] = h1
  xw2_ref[...] = jnp.dot(h1 * _scale_from(dego_ref), w2_ref[...],
                         preferred_element_type=jnp.float32)


def _fin_body(p_ref, degi_ref, b2_ref, h2_ref):
  agg = (p_ref[0] + p_ref[1]) * _scale_from(degi_ref)
  h2_ref[...] = jnp.maximum(agg + b2_ref[...], 0.0)


def _row_spec(d):
  return pl.BlockSpec((_BM, d), lambda i: (i, 0))


def _deg_spec():
  return pl.BlockSpec((_NC, _BM, 8), lambda i: (0, i, 0))


def _full_spec(shape):
  return pl.BlockSpec(shape, lambda i: tuple(0 for _ in shape))


_mm1 = pl.pallas_call(
    _mm1_body,
    grid=(_GRID,),
    in_specs=[_row_spec(_D_IN), _full_spec((_D_IN, _D_H)), _deg_spec()],
    out_specs=_row_spec(_D_H),
    out_shape=jax.ShapeDtypeStruct((_N_PAD, _D_H), jnp.float32),
)

_mid = pl.pallas_call(
    _mid_body,
    grid=(_GRID,),
    in_specs=[
        pl.BlockSpec((_NC, _BM, _D_H), lambda i: (0, i, 0)),
        _deg_spec(),
        _deg_spec(),
        _full_spec((1, _D_H)),
        _full_spec((_D_H, _D_OUT)),
    ],
    out_specs=(_row_spec(_D_H), _row_spec(_D_OUT)),
    out_shape=(
        jax.ShapeDtypeStruct((_N_PAD, _D_H), jnp.float32),
        jax.ShapeDtypeStruct((_N_PAD, _D_OUT), jnp.float32),
    ),
)

_fin = pl.pallas_call(
    _fin_body,
    grid=(_GRID,),
    in_specs=[
        pl.BlockSpec((_NC, _BM, _D_OUT), lambda i: (0, i, 0)),
        _deg_spec(),
        _full_spec((1, _D_OUT)),
    ],
    out_specs=_row_spec(_D_OUT),
    out_shape=jax.ShapeDtypeStruct((_N_PAD, _D_OUT), jnp.float32),
)


def kernel(edge_index, in_feat, W1, b1, W2, b2):
  src = edge_index[0]
  dst = edge_index[1]
  pad = jnp.full((_E_PAD - _E,), _N, jnp.int32)
  src2d = jnp.concatenate([src, pad]).reshape(_EROWS, 128)
  dst2d = jnp.concatenate([dst, pad]).reshape(_EROWS, 128)
  x_pad = jnp.zeros((_N_PAD, _D_IN), jnp.float32).at[:_N].set(in_feat)

  ones8 = jnp.ones((128, 8), jnp.float32)
  zeros8 = jnp.zeros((_N_PAD, 8), jnp.float32)
  zeros_h = jnp.zeros((_N_PAD, _D_H), jnp.float32)
  zeros_o = jnp.zeros((_N_PAD, _D_OUT), jnp.float32)

  dego, degi = _deg(src2d, dst2d, ones8, zeros8)
  xw1 = _mm1(x_pad, W1, dego)
  p1 = _prop_h(src2d, dst2d, xw1, zeros_h)
  h1, xw2 = _mid(p1, degi, dego, b1.reshape(1, _D_H), W2)
  p2 = _prop_o(src2d, dst2d, xw2, zeros_o)
  h2 = _fin(p2, degi, b2.reshape(1, _D_OUT))
  return h1[:_N], h2[:_N]


# speedup vs baseline: 2.2231x; 2.2231x over previous
import jax
import jax.numpy as jnp
from jax import lax
from jax.experimental import pallas as pl
from jax.experimental.pallas import tpu as pltpu
from jax.experimental.pallas import tpu_sc as plsc

_N = 10000
_E = 320000
_D_IN = 128
_D_H = 128
_D_OUT = 64

_NC = 2
_NS = 16
_NW = _NC * _NS

_N_PAD = 10240
_EROWS = 2560
_E_PAD = _EROWS * 128
_EROWS_W = _EROWS // _NW
_NROWS_S = _N_PAD // _NS

_BM = 1024
_GRID = _N_PAD // _BM
_BE = 2048
_EGRID = _E_PAD // _BE

_mesh = plsc.VectorSubcoreMesh(core_axis_name="c", subcore_axis_name="s")


def _prop_body(src_hbm, dst_hbm, xw_hbm, zeros_hbm, out_hbm,
               sidx, didx, rows, acc, sem):
  c = lax.axis_index("c")
  s = lax.axis_index("s")
  w = s * _NC + c
  r0 = s * _NROWS_S
  pltpu.sync_copy(zeros_hbm.at[pl.ds(r0, _NROWS_S)], acc.at[pl.ds(r0, _NROWS_S)])
  plsc.subcore_barrier()
  row0 = w * _EROWS_W

  @pl.loop(0, _EROWS_W // 8)
  def _(i):
    base = row0 + i * 8
    pltpu.sync_copy(src_hbm.at[pl.ds(base, 8)], sidx)
    pltpu.sync_copy(dst_hbm.at[pl.ds(base, 8)], didx)
    for j in range(8):
      pltpu.async_copy(xw_hbm.at[sidx.at[j]], rows, sem).wait()
      pltpu.sync_copy(rows, acc.at[didx.at[j]], add=True)

  plsc.subcore_barrier()
  pltpu.sync_copy(acc.at[pl.ds(r0, _NROWS_S)], out_hbm.at[c, pl.ds(r0, _NROWS_S)])


def _prop_types(d):
  return dict(
      out_type=jax.ShapeDtypeStruct((_NC, _N_PAD, d), jnp.float32),
      mesh=_mesh,
      scratch_types=[
          pltpu.VMEM((8, 128), jnp.int32),
          pltpu.VMEM((8, 128), jnp.int32),
          pltpu.VMEM((128, d), jnp.float32),
          pltpu.VMEM_SHARED((_N_PAD, d), jnp.float32),
          pltpu.SemaphoreType.DMA,
      ],
  )


_prop_h = pl.kernel(_prop_body, **_prop_types(_D_H))


def _onehot_pair(idx):
  a = idx >> 7
  b = idx & 127
  lanes = lax.broadcasted_iota(jnp.int32, (_BE, 128), 1)
  A = (lanes == a).astype(jnp.float32)
  B = (lanes == b).astype(jnp.float32)
  return A, B


def _bc_body(s_ref, d_ref, dego_ref, degi_ref):
  i = pl.program_id(0)

  @pl.when(i == 0)
  def _():
    dego_ref[...] = jnp.zeros_like(dego_ref)
    degi_ref[...] = jnp.zeros_like(degi_ref)

  dn = (((0,), (0,)), ((), ()))
  As, Bs = _onehot_pair(s_ref[...])
  dego_ref[...] += lax.dot_general(As, Bs, dn,
                                   preferred_element_type=jnp.float32)
  Ad, Bd = _onehot_pair(d_ref[...])
  degi_ref[...] += lax.dot_general(Ad, Bd, dn,
                                   preferred_element_type=jnp.float32)


_bc = pl.pallas_call(
    _bc_body,
    grid=(_EGRID,),
    in_specs=[
        pl.BlockSpec((_BE, 1), lambda i: (i, 0)),
        pl.BlockSpec((_BE, 1), lambda i: (i, 0)),
    ],
    out_specs=(
        pl.BlockSpec((128, 128), lambda i: (0, 0)),
        pl.BlockSpec((128, 128), lambda i: (0, 0)),
    ),
    out_shape=(
        jax.ShapeDtypeStruct((128, 128), jnp.float32),
        jax.ShapeDtypeStruct((128, 128), jnp.float32),
    ),
)


def _scale_from(deg_ref):
  return lax.rsqrt(jnp.maximum(deg_ref[...], 1.0))


def _mm1_body(x_ref, w_ref, dego_ref, o_ref):
  xw = jnp.dot(x_ref[...], w_ref[...], preferred_element_type=jnp.float32)
  o_ref[...] = _scale_from(dego_ref) * xw


def _mid_body(p_ref, degi_ref, dego_ref, b1_ref, w2_ref, h1_ref, xw2_ref):
  agg = (p_ref[0] + p_ref[1]) * _scale_from(degi_ref)
  h1 = jnp.maximum(agg + b1_ref[...], 0.0)
  h1_ref[...] = h1
  xw2_ref[...] = jnp.dot(h1 * _scale_from(dego_ref), w2_ref[...],
                         preferred_element_type=jnp.float32)


def _fin_body(p_ref, degi_ref, b2_ref, h2_ref):
  agg = (p_ref[0][:, :_D_OUT] + p_ref[1][:, :_D_OUT]) * _scale_from(degi_ref)
  h2_ref[...] = jnp.maximum(agg + b2_ref[...], 0.0)


def _row_spec(d):
  return pl.BlockSpec((_BM, d), lambda i: (i, 0))


def _deg_spec():
  return pl.BlockSpec((_BM, 1), lambda i: (i, 0))


def _full_spec(shape):
  return pl.BlockSpec(shape, lambda i: tuple(0 for _ in shape))


_mm1 = pl.pallas_call(
    _mm1_body,
    grid=(_GRID,),
    in_specs=[_row_spec(_D_IN), _full_spec((_D_IN, _D_H)), _deg_spec()],
    out_specs=_row_spec(_D_H),
    out_shape=jax.ShapeDtypeStruct((_N_PAD, _D_H), jnp.float32),
)

_mid = pl.pallas_call(
    _mid_body,
    grid=(_GRID,),
    in_specs=[
        pl.BlockSpec((_NC, _BM, _D_H), lambda i: (0, i, 0)),
        _deg_spec(),
        _deg_spec(),
        _full_spec((1, _D_H)),
        _full_spec((_D_H, _D_H)),
    ],
    out_specs=(_row_spec(_D_H), _row_spec(_D_H)),
    out_shape=(
        jax.ShapeDtypeStruct((_N_PAD, _D_H), jnp.float32),
        jax.ShapeDtypeStruct((_N_PAD, _D_H), jnp.float32),
    ),
)

_fin = pl.pallas_call(
    _fin_body,
    grid=(_GRID,),
    in_specs=[
        pl.BlockSpec((_NC, _BM, _D_H), lambda i: (0, i, 0)),
        _deg_spec(),
        _full_spec((1, _D_OUT)),
    ],
    out_specs=_row_spec(_D_OUT),
    out_shape=jax.ShapeDtypeStruct((_N_PAD, _D_OUT), jnp.float32),
)


def kernel(edge_index, in_feat, W1, b1, W2, b2):
  src = edge_index[0]
  dst = edge_index[1]
  pad = jnp.full((_E_PAD - _E,), _N, jnp.int32)
  src_p = jnp.concatenate([src, pad])
  dst_p = jnp.concatenate([dst, pad])
  src2d = src_p.reshape(_EROWS, 128)
  dst2d = dst_p.reshape(_EROWS, 128)
  x_pad = jnp.zeros((_N_PAD, _D_IN), jnp.float32).at[:_N].set(in_feat)

  zeros_h = jnp.zeros((_N_PAD, _D_H), jnp.float32)
  W2p = jnp.zeros((_D_H, _D_H), jnp.float32).at[:, :_D_OUT].set(W2)

  dego128, degi128 = _bc(src_p.reshape(_E_PAD, 1), dst_p.reshape(_E_PAD, 1))
  deg_o = dego128.reshape(-1)[:_N_PAD].reshape(_N_PAD, 1)
  deg_i = degi128.reshape(-1)[:_N_PAD].reshape(_N_PAD, 1)

  xw1 = _mm1(x_pad, W1, deg_o)
  p1 = _prop_h(src2d, dst2d, xw1, zeros_h)
  h1, xw2 = _mid(p1, deg_i, deg_o, b1.reshape(1, _D_H), W2p)
  p2 = _prop_h(src2d, dst2d, xw2, zeros_h)
  h2 = _fin(p2, deg_i, b2.reshape(1, _D_OUT))
  return h1[:_N], h2[:_N]


# speedup vs baseline: 2.4111x; 1.0846x over previous
import jax
import jax.numpy as jnp
from jax import lax
from jax.experimental import pallas as pl
from jax.experimental.pallas import tpu as pltpu
from jax.experimental.pallas import tpu_sc as plsc

_N = 10000
_E = 320000
_D_IN = 128
_D_H = 128
_D_OUT = 64

_NC = 2
_NS = 16
_NW = _NC * _NS

_N_PAD = 10240
_EROWS = 2560
_E_PAD = _EROWS * 128
_EROWS_W = _EROWS // _NW
_NROWS_S = _N_PAD // _NS

_BM = 1024
_GRID = _N_PAD // _BM
_BE = 2048
_EGRID = _E_PAD // _BE

_mesh = plsc.VectorSubcoreMesh(core_axis_name="c", subcore_axis_name="s")


_NBUF = 2
_SLAB = 16


def _prop_body(src_hbm, dst_hbm, xw_hbm, zeros_hbm, out_hbm,
               sidx, didx, rows, acc, *sems):
  c = lax.axis_index("c")
  s = lax.axis_index("s")
  w = s * _NC + c
  r0 = s * _NROWS_S
  pltpu.sync_copy(zeros_hbm.at[pl.ds(r0, _NROWS_S)], acc.at[pl.ds(r0, _NROWS_S)])
  row0 = w * _EROWS_W
  plsc.subcore_barrier()

  @pl.loop(0, _EROWS_W // _SLAB)
  def _(sl):
    base = row0 + sl * _SLAB
    pltpu.sync_copy(src_hbm.at[pl.ds(base, _SLAB)], sidx)
    pltpu.sync_copy(dst_hbm.at[pl.ds(base, _SLAB)], didx)
    gathers = [
        pltpu.async_copy(xw_hbm.at[sidx.at[b]], rows.at[b], sems[b])
        for b in range(_NBUF)
    ]

    @pl.loop(0, _SLAB // _NBUF)
    def _(i):
      for b in range(_NBUF):
        k = i * _NBUF + b
        gathers[b].wait()
        pltpu.sync_copy(rows.at[b], acc.at[didx.at[k]], add=True)

        @pl.when(k + _NBUF < _SLAB)
        def _():
          pltpu.async_copy(xw_hbm.at[sidx.at[k + _NBUF]], rows.at[b], sems[b])

  plsc.subcore_barrier()
  pltpu.sync_copy(acc.at[pl.ds(r0, _NROWS_S)], out_hbm.at[c, pl.ds(r0, _NROWS_S)])


def _prop_types(d):
  return dict(
      out_type=jax.ShapeDtypeStruct((_NC, _N_PAD, d), jnp.float32),
      mesh=_mesh,
      scratch_types=[
          pltpu.VMEM((_SLAB, 128), jnp.int32),
          pltpu.VMEM((_SLAB, 128), jnp.int32),
          pltpu.VMEM((_NBUF, 128, d), jnp.float32),
          pltpu.VMEM_SHARED((_N_PAD, d), jnp.float32),
      ] + [pltpu.SemaphoreType.DMA] * _NBUF,
  )


_prop_h = pl.kernel(_prop_body, **_prop_types(_D_H))


def _onehot_pair(idx):
  a = idx >> 7
  b = idx & 127
  lanes = lax.broadcasted_iota(jnp.int32, (_BE, 128), 1)
  A = (lanes == a).astype(jnp.float32)
  B = (lanes == b).astype(jnp.float32)
  return A, B


def _bc_body(s_ref, d_ref, dego_ref, degi_ref):
  i = pl.program_id(0)

  @pl.when(i == 0)
  def _():
    dego_ref[...] = jnp.zeros_like(dego_ref)
    degi_ref[...] = jnp.zeros_like(degi_ref)

  dn = (((0,), (0,)), ((), ()))
  As, Bs = _onehot_pair(s_ref[...])
  dego_ref[...] += lax.dot_general(As, Bs, dn,
                                   preferred_element_type=jnp.float32)
  Ad, Bd = _onehot_pair(d_ref[...])
  degi_ref[...] += lax.dot_general(Ad, Bd, dn,
                                   preferred_element_type=jnp.float32)


_bc = pl.pallas_call(
    _bc_body,
    grid=(_EGRID,),
    in_specs=[
        pl.BlockSpec((_BE, 1), lambda i: (i, 0)),
        pl.BlockSpec((_BE, 1), lambda i: (i, 0)),
    ],
    out_specs=(
        pl.BlockSpec((128, 128), lambda i: (0, 0)),
        pl.BlockSpec((128, 128), lambda i: (0, 0)),
    ),
    out_shape=(
        jax.ShapeDtypeStruct((128, 128), jnp.float32),
        jax.ShapeDtypeStruct((128, 128), jnp.float32),
    ),
)


def _scale_from(deg_ref):
  return lax.rsqrt(jnp.maximum(deg_ref[...], 1.0))


def _mm1_body(x_ref, w_ref, dego_ref, o_ref):
  xw = jnp.dot(x_ref[...], w_ref[...], preferred_element_type=jnp.float32)
  o_ref[...] = _scale_from(dego_ref) * xw


def _mid_body(p_ref, degi_ref, dego_ref, b1_ref, w2_ref, h1_ref, xw2_ref):
  agg = (p_ref[0] + p_ref[1]) * _scale_from(degi_ref)
  h1 = jnp.maximum(agg + b1_ref[...], 0.0)
  h1_ref[...] = h1
  xw2_ref[...] = jnp.dot(h1 * _scale_from(dego_ref), w2_ref[...],
                         preferred_element_type=jnp.float32)


def _fin_body(p_ref, degi_ref, b2_ref, h2_ref):
  agg = (p_ref[0][:, :_D_OUT] + p_ref[1][:, :_D_OUT]) * _scale_from(degi_ref)
  h2_ref[...] = jnp.maximum(agg + b2_ref[...], 0.0)


def _row_spec(d):
  return pl.BlockSpec((_BM, d), lambda i: (i, 0))


def _deg_spec():
  return pl.BlockSpec((_BM, 1), lambda i: (i, 0))


def _full_spec(shape):
  return pl.BlockSpec(shape, lambda i: tuple(0 for _ in shape))


_mm1 = pl.pallas_call(
    _mm1_body,
    grid=(_GRID,),
    in_specs=[_row_spec(_D_IN), _full_spec((_D_IN, _D_H)), _deg_spec()],
    out_specs=_row_spec(_D_H),
    out_shape=jax.ShapeDtypeStruct((_N_PAD, _D_H), jnp.float32),
)

_mid = pl.pallas_call(
    _mid_body,
    grid=(_GRID,),
    in_specs=[
        pl.BlockSpec((_NC, _BM, _D_H), lambda i: (0, i, 0)),
        _deg_spec(),
        _deg_spec(),
        _full_spec((1, _D_H)),
        _full_spec((_D_H, _D_H)),
    ],
    out_specs=(_row_spec(_D_H), _row_spec(_D_H)),
    out_shape=(
        jax.ShapeDtypeStruct((_N_PAD, _D_H), jnp.float32),
        jax.ShapeDtypeStruct((_N_PAD, _D_H), jnp.float32),
    ),
)

_fin = pl.pallas_call(
    _fin_body,
    grid=(_GRID,),
    in_specs=[
        pl.BlockSpec((_NC, _BM, _D_H), lambda i: (0, i, 0)),
        _deg_spec(),
        _full_spec((1, _D_OUT)),
    ],
    out_specs=_row_spec(_D_OUT),
    out_shape=jax.ShapeDtypeStruct((_N_PAD, _D_OUT), jnp.float32),
)


def kernel(edge_index, in_feat, W1, b1, W2, b2):
  src = edge_index[0]
  dst = edge_index[1]
  pad = jnp.full((_E_PAD - _E,), _N, jnp.int32)
  src_p = jnp.concatenate([src, pad])
  dst_p = jnp.concatenate([dst, pad])
  src2d = src_p.reshape(_EROWS, 128)
  dst2d = dst_p.reshape(_EROWS, 128)
  x_pad = jnp.zeros((_N_PAD, _D_IN), jnp.float32).at[:_N].set(in_feat)

  zeros_h = jnp.zeros((_N_PAD, _D_H), jnp.float32)
  W2p = jnp.zeros((_D_H, _D_H), jnp.float32).at[:, :_D_OUT].set(W2)

  dego128, degi128 = _bc(src_p.reshape(_E_PAD, 1), dst_p.reshape(_E_PAD, 1))
  deg_o = dego128.reshape(-1)[:_N_PAD].reshape(_N_PAD, 1)
  deg_i = degi128.reshape(-1)[:_N_PAD].reshape(_N_PAD, 1)

  xw1 = _mm1(x_pad, W1, deg_o)
  p1 = _prop_h(src2d, dst2d, xw1, zeros_h)
  h1, xw2 = _mid(p1, deg_i, deg_o, b1.reshape(1, _D_H), W2p)
  p2 = _prop_h(src2d, dst2d, xw2, zeros_h)
  h2 = _fin(p2, deg_i, b2.reshape(1, _D_OUT))
  return h1[:_N], h2[:_N]


# speedup vs baseline: 2.6444x; 1.0967x over previous
import jax
import jax.numpy as jnp
from jax import lax
from jax.experimental import pallas as pl
from jax.experimental.pallas import tpu as pltpu
from jax.experimental.pallas import tpu_sc as plsc

_N = 10000
_E = 320000
_D_IN = 128
_D_H = 128
_D_OUT = 64

_NC = 2
_NS = 16
_NW = _NC * _NS

_N_PAD = 10240
_EROWS = 2560
_E_PAD = _EROWS * 128
_EROWS_W = _EROWS // _NW
_NROWS_S = _N_PAD // _NS

_BM = 1024
_GRID = _N_PAD // _BM
_BE = 2048
_EGRID = _E_PAD // _BE

_mesh = plsc.VectorSubcoreMesh(core_axis_name="c", subcore_axis_name="s")


_NBUF = 2
_SLAB = 16
_ROWS_C0 = 112
_ROWS_C1 = 2 * _EROWS_W - _ROWS_C0


def _prop_body(src_hbm, dst_hbm, xw_hbm, zeros_hbm, out_hbm,
               sidx, didx, rows, acc, *sems):
  c = lax.axis_index("c")
  s = lax.axis_index("s")
  r0 = s * _NROWS_S
  pltpu.sync_copy(zeros_hbm.at[pl.ds(r0, _NROWS_S)], acc.at[pl.ds(r0, _NROWS_S)])
  pair0 = s * (_ROWS_C0 + _ROWS_C1)
  row0 = pair0 + jnp.where(c == 0, 0, _ROWS_C0)
  nslabs = jnp.where(c == 0, _ROWS_C0 // _SLAB, _ROWS_C1 // _SLAB)
  plsc.subcore_barrier()

  @pl.loop(0, nslabs)
  def _(sl):
      base = row0 + sl * _SLAB
      pltpu.sync_copy(src_hbm.at[pl.ds(base, _SLAB)], sidx)
      pltpu.sync_copy(dst_hbm.at[pl.ds(base, _SLAB)], didx)
      gathers = [
          pltpu.async_copy(xw_hbm.at[sidx.at[b]], rows.at[b], sems[b])
          for b in range(_NBUF)
      ]

      @pl.loop(0, _SLAB // _NBUF)
      def _(i):
        for b in range(_NBUF):
          k = i * _NBUF + b
          gathers[b].wait()
          pltpu.sync_copy(rows.at[b], acc.at[didx.at[k]], add=True)

          @pl.when(k + _NBUF < _SLAB)
          def _():
            pltpu.async_copy(xw_hbm.at[sidx.at[k + _NBUF]], rows.at[b], sems[b])

  plsc.subcore_barrier()
  pltpu.sync_copy(acc.at[pl.ds(r0, _NROWS_S)], out_hbm.at[c, pl.ds(r0, _NROWS_S)])


def _prop_types(d):
  return dict(
      out_type=jax.ShapeDtypeStruct((_NC, _N_PAD, d), jnp.float32),
      mesh=_mesh,
      scratch_types=[
          pltpu.VMEM((_SLAB, 128), jnp.int32),
          pltpu.VMEM((_SLAB, 128), jnp.int32),
          pltpu.VMEM((_NBUF, 128, d), jnp.float32),
          pltpu.VMEM_SHARED((_N_PAD, d), jnp.float32),
      ] + [pltpu.SemaphoreType.DMA] * _NBUF,
  )


_prop_h = pl.kernel(_prop_body, **_prop_types(_D_H))


def _onehot_pair(idx):
  a = idx >> 7
  b = idx & 127
  lanes = lax.broadcasted_iota(jnp.int32, (_BE, 128), 1)
  A = (lanes == a).astype(jnp.float32)
  B = (lanes == b).astype(jnp.float32)
  return A, B


def _bc_body(s_ref, d_ref, dego_ref, degi_ref):
  i = pl.program_id(0)

  @pl.when(i == 0)
  def _():
    dego_ref[...] = jnp.zeros_like(dego_ref)
    degi_ref[...] = jnp.zeros_like(degi_ref)

  dn = (((0,), (0,)), ((), ()))
  As, Bs = _onehot_pair(s_ref[...])
  dego_ref[...] += lax.dot_general(As, Bs, dn,
                                   preferred_element_type=jnp.float32)
  Ad, Bd = _onehot_pair(d_ref[...])
  degi_ref[...] += lax.dot_general(Ad, Bd, dn,
                                   preferred_element_type=jnp.float32)


_bc = pl.pallas_call(
    _bc_body,
    grid=(_EGRID,),
    in_specs=[
        pl.BlockSpec((_BE, 1), lambda i: (i, 0)),
        pl.BlockSpec((_BE, 1), lambda i: (i, 0)),
    ],
    out_specs=(
        pl.BlockSpec((128, 128), lambda i: (0, 0)),
        pl.BlockSpec((128, 128), lambda i: (0, 0)),
    ),
    out_shape=(
        jax.ShapeDtypeStruct((128, 128), jnp.float32),
        jax.ShapeDtypeStruct((128, 128), jnp.float32),
    ),
)


def _scale_from(deg_ref):
  return lax.rsqrt(jnp.maximum(deg_ref[...], 1.0))


def _mm1_body(x_ref, w_ref, dego_ref, o_ref):
  xw = jnp.dot(x_ref[...], w_ref[...], preferred_element_type=jnp.float32)
  o_ref[...] = _scale_from(dego_ref) * xw


def _mid_body(p_ref, degi_ref, dego_ref, b1_ref, w2_ref, h1_ref, xw2_ref):
  agg = (p_ref[0] + p_ref[1]) * _scale_from(degi_ref)
  h1 = jnp.maximum(agg + b1_ref[...], 0.0)
  h1_ref[...] = h1
  xw2_ref[...] = jnp.dot(h1 * _scale_from(dego_ref), w2_ref[...],
                         preferred_element_type=jnp.float32)


def _fin_body(p_ref, degi_ref, b2_ref, h2_ref):
  agg = (p_ref[0][:, :_D_OUT] + p_ref[1][:, :_D_OUT]) * _scale_from(degi_ref)
  h2_ref[...] = jnp.maximum(agg + b2_ref[...], 0.0)


def _row_spec(d):
  return pl.BlockSpec((_BM, d), lambda i: (i, 0))


def _deg_spec():
  return pl.BlockSpec((_BM, 1), lambda i: (i, 0))


def _full_spec(shape):
  return pl.BlockSpec(shape, lambda i: tuple(0 for _ in shape))


_mm1 = pl.pallas_call(
    _mm1_body,
    grid=(_GRID,),
    in_specs=[_row_spec(_D_IN), _full_spec((_D_IN, _D_H)), _deg_spec()],
    out_specs=_row_spec(_D_H),
    out_shape=jax.ShapeDtypeStruct((_N_PAD, _D_H), jnp.float32),
)

_mid = pl.pallas_call(
    _mid_body,
    grid=(_GRID,),
    in_specs=[
        pl.BlockSpec((_NC, _BM, _D_H), lambda i: (0, i, 0)),
        _deg_spec(),
        _deg_spec(),
        _full_spec((1, _D_H)),
        _full_spec((_D_H, _D_H)),
    ],
    out_specs=(_row_spec(_D_H), _row_spec(_D_H)),
    out_shape=(
        jax.ShapeDtypeStruct((_N_PAD, _D_H), jnp.float32),
        jax.ShapeDtypeStruct((_N_PAD, _D_H), jnp.float32),
    ),
)

_fin = pl.pallas_call(
    _fin_body,
    grid=(_GRID,),
    in_specs=[
        pl.BlockSpec((_NC, _BM, _D_H), lambda i: (0, i, 0)),
        _deg_spec(),
        _full_spec((1, _D_OUT)),
    ],
    out_specs=_row_spec(_D_OUT),
    out_shape=jax.ShapeDtypeStruct((_N_PAD, _D_OUT), jnp.float32),
)


def kernel(edge_index, in_feat, W1, b1, W2, b2):
  src = edge_index[0]
  dst = edge_index[1]
  pad = jnp.full((_E_PAD - _E,), _N, jnp.int32)
  src_p = jnp.concatenate([src, pad])
  dst_p = jnp.concatenate([dst, pad])
  src2d = src_p.reshape(_EROWS, 128)
  dst2d = dst_p.reshape(_EROWS, 128)
  x_pad = jnp.zeros((_N_PAD, _D_IN), jnp.float32).at[:_N].set(in_feat)

  zeros_h = jnp.zeros((_N_PAD, _D_H), jnp.float32)
  W2p = jnp.zeros((_D_H, _D_H), jnp.float32).at[:, :_D_OUT].set(W2)

  dego128, degi128 = _bc(src_p.reshape(_E_PAD, 1), dst_p.reshape(_E_PAD, 1))
  deg_o = dego128.reshape(-1)[:_N_PAD].reshape(_N_PAD, 1)
  deg_i = degi128.reshape(-1)[:_N_PAD].reshape(_N_PAD, 1)

  xw1 = _mm1(x_pad, W1, deg_o)
  p1 = _prop_h(src2d, dst2d, xw1, zeros_h)
  h1, xw2 = _mid(p1, deg_i, deg_o, b1.reshape(1, _D_H), W2p)
  p2 = _prop_h(src2d, dst2d, xw2, zeros_h)
  h2 = _fin(p2, deg_i, b2.reshape(1, _D_OUT))
  return h1[:_N], h2[:_N]
